# baseline (speedup 1.0000x reference)
import functools

import jax
import jax.numpy as jnp
from jax import lax
from jax.experimental import pallas as pl
from jax.experimental.pallas import tpu as pltpu
from jax.experimental.pallas import tpu_sc as plsc

D = 64
NC, NS = 2, 16
NW = NC * NS
BB = 128
TB = 8
NBUF = 4


def _make_lookup(t_len: int, b_len: int):
    assert b_len == NW * BB and t_len % TB == 0 and (t_len // TB) % 2 == 1
    ntt = t_len // TB

    mesh = plsc.VectorSubcoreMesh(core_axis_name="c", subcore_axis_name="s")

    @functools.partial(
        pl.kernel,
        out_type=jax.ShapeDtypeStruct((t_len, D // TB, NW, TB, BB), jnp.float32),
        mesh=mesh,
        scratch_types=[
            pltpu.VMEM((TB, BB), jnp.int32),
            pltpu.VMEM((TB, BB), jnp.int32),
            pltpu.VMEM((NBUF, BB, D), jnp.float32),
            pltpu.VMEM((2, D // TB, TB, BB), jnp.float32),
            pltpu.SemaphoreType.DMA,
            pltpu.SemaphoreType.DMA,
            pltpu.SemaphoreType.DMA((NBUF,)),
            pltpu.SemaphoreType.DMA((2,)),
        ],
        compiler_params=pltpu.CompilerParams(
            use_tc_tiling_on_sc=False, needs_layout_passes=False
        ),
    )
    def lookup(xt_hbm, table_hbm, out_hbm, xa, xb, g_v, tr_v, sema, semb,
               gsem, osem):
        w = lax.axis_index("s") * NC + lax.axis_index("c")
        b0 = w * BB
        lanes = lax.iota(jnp.int32, 16)

        def xsrc(tt):
            return xt_hbm.at[pl.ds(tt * TB, TB), pl.ds(b0, BB)]

        def process(tt, xv, xs, pre):
            pltpu.make_async_copy(xsrc(tt), xv, xs).wait()
            if pre is not None:
                ptt, pxv, pxs = pre
                pltpu.async_copy(xsrc(ptt), pxv, pxs)
            gw = [None] * TB
            ow = [None, None]
            for s in range(min(NBUF, TB)):
                gw[s] = pltpu.async_copy(
                    table_hbm.at[xv.at[s]], g_v.at[s % NBUF], gsem.at[s % NBUF]
                )
            for s in range(TB):
                gw[s].wait()
                if ow[s % 2] is not None:
                    ow[s % 2].wait()
                g = g_v.at[s % NBUF]
                tr = tr_v.at[s % 2]

                @pl.loop(0, D)
                def _col(c):
                    k = lax.shift_right_logical(c, 3)
                    s8 = lax.bitwise_and(c, 7)
                    cvec = jnp.zeros((16,), jnp.int32) + c
                    for i in range(BB // 16):
                        vec = plsc.load_gather(g, [i * 16 + lanes, cvec])
                        tr[k, s8, pl.ds(i * 16, 16)] = vec

                ow[s % 2] = pltpu.async_copy(
                    tr, out_hbm.at[tt * TB + s, :, w], osem.at[s % 2]
                )
                if s + NBUF < TB:
                    gw[s + NBUF] = pltpu.async_copy(
                        table_hbm.at[xv.at[s + NBUF]],
                        g_v.at[(s + NBUF) % NBUF],
                        gsem.at[(s + NBUF) % NBUF],
                    )
            ow[0].wait()
            ow[1].wait()

        pltpu.async_copy(xsrc(0), xa, sema)

        @pl.loop(0, ntt // 2)
        def _pair(p):
            tt0 = 2 * p
            process(tt0, xa, sema, (tt0 + 1, xb, semb))
            process(tt0 + 1, xb, semb, (tt0 + 2, xa, sema))

        process(ntt - 1, xa, sema, None)

    return lookup


def kernel(x, emb_weight):
    b, t = x.shape
    out5 = _make_lookup(t, b)(x.T.astype(jnp.int32), emb_weight)
    return out5.transpose(2, 4, 0, 1, 3).reshape(b, t, D)


# speedup vs baseline: 1.5781x; 1.5781x over previous
import functools

import jax
import jax.numpy as jnp
from jax import lax
from jax.experimental import pallas as pl
from jax.experimental.pallas import tpu as pltpu
from jax.experimental.pallas import tpu_sc as plsc

D = 64
NC, NS = 2, 16
NW = NC * NS
BB = 128
TB = 8
NBUF = 4
VB = 4096


def _prep_block(wt_ref, out_ref):
    sw = jnp.swapaxes(wt_ref[...], 0, 1)
    sw3 = sw.reshape(VB // 2, 2, D)
    out_ref[:, 0:D] = sw3[:, 0, :]
    out_ref[:, D : 2 * D] = sw3[:, 1, :]


def _tc_prep(wt):
    d, v = wt.shape
    return pl.pallas_call(
        _prep_block,
        grid=((v + VB - 1) // VB,),
        in_specs=[pl.BlockSpec((d, VB), lambda i: (0, i))],
        out_specs=pl.BlockSpec((VB // 2, 2 * d), lambda i: (i, 0)),
        out_shape=jax.ShapeDtypeStruct((v // 2, 2 * d), jnp.float32),
    )(wt)


def _make_lookup(t_len: int, b_len: int):
    assert b_len == NW * BB and t_len % TB == 0 and (t_len // TB) % 2 == 1
    ntt = t_len // TB

    mesh = plsc.VectorSubcoreMesh(core_axis_name="c", subcore_axis_name="s")

    @functools.partial(
        pl.kernel,
        out_type=jax.ShapeDtypeStruct((t_len, D // TB, NW, TB, BB), jnp.float32),
        mesh=mesh,
        scratch_types=[
            pltpu.VMEM((TB, BB), jnp.int32),
            pltpu.VMEM((TB, BB), jnp.int32),
            pltpu.VMEM((TB, BB), jnp.int32),
            pltpu.VMEM((TB, BB), jnp.int32),
            pltpu.VMEM((NBUF, BB, 2 * D), jnp.float32),
            pltpu.VMEM((2, D // TB, TB, BB), jnp.float32),
            pltpu.SemaphoreType.DMA,
            pltpu.SemaphoreType.DMA,
            pltpu.SemaphoreType.DMA((NBUF,)),
            pltpu.SemaphoreType.DMA((2,)),
        ],
        compiler_params=pltpu.CompilerParams(
            use_tc_tiling_on_sc=False, needs_layout_passes=False
        ),
    )
    def lookup(xt_hbm, table_hbm, out_hbm, xa, xb, pa, pb, g_v, tr_v,
               sema, semb, gsem, osem):
        w = lax.axis_index("s") * NC + lax.axis_index("c")
        b0 = w * BB
        lanes = lax.iota(jnp.int32, 16)

        def xsrc(tt):
            return xt_hbm.at[pl.ds(tt * TB, TB), pl.ds(b0, BB)]

        def process(tt, xv, pv, xs, pre):
            pltpu.make_async_copy(xsrc(tt), xv, xs).wait()
            if pre is not None:
                ptt, pxv, pxs = pre
                pltpu.async_copy(xsrc(ptt), pxv, pxs)
            for s in range(TB):
                for ch in range(0, BB, 16):
                    pv[s, pl.ds(ch, 16)] = lax.shift_right_logical(
                        xv[s, pl.ds(ch, 16)], 1
                    )
            gw = [None] * TB
            ow = [None, None]
            for s in range(min(NBUF, TB)):
                gw[s] = pltpu.async_copy(
                    table_hbm.at[pv.at[s]], g_v.at[s % NBUF], gsem.at[s % NBUF]
                )
            for s in range(TB):
                gw[s].wait()
                if ow[s % 2] is not None:
                    ow[s % 2].wait()
                g = g_v.at[s % NBUF]
                tr = tr_v.at[s % 2]
                hv = [
                    lax.shift_left(
                        lax.bitwise_and(xv[s, pl.ds(i * 16, 16)], 1), 6
                    )
                    for i in range(BB // 16)
                ]

                @plsc.parallel_loop(0, D, unroll=4)
                def _col(c):
                    k = lax.shift_right_logical(c, 3)
                    s8 = lax.bitwise_and(c, 7)
                    cvec = jnp.zeros((16,), jnp.int32) + c
                    for i in range(BB // 16):
                        vec = plsc.load_gather(
                            g, [i * 16 + lanes, hv[i] + cvec]
                        )
                        tr[k, s8, pl.ds(i * 16, 16)] = vec

                ow[s % 2] = pltpu.async_copy(
                    tr, out_hbm.at[tt * TB + s, :, w], osem.at[s % 2]
                )
                if s + NBUF < TB:
                    gw[s + NBUF] = pltpu.async_copy(
                        table_hbm.at[pv.at[s + NBUF]],
                        g_v.at[(s + NBUF) % NBUF],
                        gsem.at[(s + NBUF) % NBUF],
                    )
            ow[0].wait()
            ow[1].wait()

        pltpu.async_copy(xsrc(0), xa, sema)

        @pl.loop(0, ntt // 2)
        def _pair(p):
            tt0 = 2 * p
            process(tt0, xa, pa, sema, (tt0 + 1, xb, semb))
            process(tt0 + 1, xb, pb, semb, (tt0 + 2, xa, sema))

        process(ntt - 1, xa, pa, sema, None)

    return lookup


def kernel(x, emb_weight):
    b, t = x.shape
    table2 = _tc_prep(emb_weight.T)
    out5 = _make_lookup(t, b)(x.T.astype(jnp.int32), table2)
    return out5.transpose(2, 4, 0, 1, 3).reshape(b, t, D)
